# Initial kernel scaffold; baseline (speedup 1.0000x reference)
import jax
import jax.numpy as jnp
from jax.experimental import pallas as pl


def kernel(atom_features, node_graph_indices):
    raise NotImplementedError("write your pallas kernel here")


# speedup vs baseline: 4.7508x; 4.7508x over previous
import jax
import jax.numpy as jnp
from jax import lax
from jax.experimental import pallas as pl
from jax.experimental.pallas import tpu as pltpu
from jax.experimental.pallas import tpu_sc as plsc

_N_ATOMS = 320000
_D = 128
_NSEG = 4096
_TILE = 128
_NT = _N_ATOMS // _TILE
_NC, _NS = 2, 16
_NW = _NC * _NS
_TPW = -(-_NT // _NW)
_RPS = _NSEG // _NS


def _sc_body(feat_hbm, ids_hbm, part_hbm, idx_v, rows_v, acc_sh):
    c = lax.axis_index("c")
    s = lax.axis_index("s")
    w = c * _NS + s

    @pl.loop(0, _TILE)
    def _zero_rows(i):
        @pl.loop(0, _D // 16)
        def _zero_vec(j):
            rows_v[i, pl.ds(j * 16, 16)] = jnp.zeros((16,), jnp.float32)

    pltpu.sync_copy(rows_v, acc_sh.at[pl.ds(s * _RPS, _TILE)])
    pltpu.sync_copy(rows_v, acc_sh.at[pl.ds(s * _RPS + _TILE, _TILE)])
    plsc.subcore_barrier()

    @pl.loop(0, _TPW)
    def _tile(i):
        t = w + i * _NW

        @pl.when(t < _NT)
        def _():
            base = t * _TILE
            pltpu.sync_copy(ids_hbm.at[pl.ds(base, _TILE)], idx_v)
            pltpu.sync_copy(feat_hbm.at[pl.ds(base, _TILE)], rows_v)
            pltpu.sync_copy(rows_v, acc_sh.at[idx_v], add=True)

    plsc.subcore_barrier()
    pltpu.sync_copy(
        acc_sh.at[pl.ds(s * _RPS, _RPS)],
        part_hbm.at[c, pl.ds(s * _RPS, _RPS)],
    )


def _add_body(p_ref, o_ref):
    o_ref[...] = p_ref[0] + p_ref[1]


def kernel(atom_features, node_graph_indices):
    ids = node_graph_indices.astype(jnp.int32)
    mesh = plsc.VectorSubcoreMesh(core_axis_name="c", subcore_axis_name="s")
    sc_call = pl.kernel(
        _sc_body,
        out_type=jax.ShapeDtypeStruct((_NC, _NSEG, _D), jnp.float32),
        mesh=mesh,
        scratch_types=[
            pltpu.VMEM((_TILE,), jnp.int32),
            pltpu.VMEM((_TILE, _D), jnp.float32),
            pltpu.VMEM_SHARED((_NSEG, _D), jnp.float32),
        ],
    )
    part = sc_call(atom_features, ids)
    return pl.pallas_call(
        _add_body,
        out_shape=jax.ShapeDtypeStruct((_NSEG, _D), jnp.float32),
    )(part)


# speedup vs baseline: 8.6963x; 1.8305x over previous
import jax
import jax.numpy as jnp
from jax import lax
from jax.experimental import pallas as pl
from jax.experimental.pallas import tpu as pltpu
from jax.experimental.pallas import tpu_sc as plsc

_N_ATOMS = 320000
_D = 128
_NSEG = 4096
_TILE = 128
_NT = _N_ATOMS // _TILE
_NC, _NS = 2, 16
_NW = _NC * _NS
_TPW = 80
_SLOTS = 80
_NBUF = 4
_RPS = _NSEG // _NS
_IDS_PAD = _NW * _TPW


def _sc_body(feat_hbm, ids2d_hbm, part_hbm, idx_v, rows_v, acc_sh, sems):
    c = lax.axis_index("c")
    s = lax.axis_index("s")
    w = c * _NS + s
    t0 = w * _TPW

    @pl.loop(0, _TILE)
    def _zero_rows(i):
        @pl.loop(0, _D // 16)
        def _zero_vec(j):
            rows_v[0, i, pl.ds(j * 16, 16)] = jnp.zeros((16,), jnp.float32)

    pltpu.sync_copy(rows_v.at[0], acc_sh.at[pl.ds(s * _RPS, _TILE)])
    pltpu.sync_copy(rows_v.at[0], acc_sh.at[pl.ds(s * _RPS + _TILE, _TILE)])
    plsc.subcore_barrier()

    pltpu.sync_copy(ids2d_hbm.at[pl.ds(t0, _TPW)], idx_v)

    for b in range(_NBUF):
        pltpu.make_async_copy(
            feat_hbm.at[pl.ds((t0 + b) * _TILE, _TILE)],
            rows_v.at[b],
            sems.at[b],
        ).start()

    @pl.loop(0, _SLOTS // _NBUF)
    def _grp(g):
        for b in range(_NBUF):
            i = g * _NBUF + b
            t = t0 + i

            @pl.when((i < _TPW) & (t < _NT))
            def _consume():
                pltpu.make_async_copy(
                    feat_hbm.at[pl.ds(t * _TILE, _TILE)],
                    rows_v.at[b],
                    sems.at[b],
                ).wait()
                pltpu.sync_copy(rows_v.at[b], acc_sh.at[idx_v.at[i]], add=True)

            i2 = i + _NBUF
            t2 = t + _NBUF

            @pl.when((i2 < _TPW) & (t2 < _NT))
            def _prefetch():
                pltpu.make_async_copy(
                    feat_hbm.at[pl.ds(t2 * _TILE, _TILE)],
                    rows_v.at[b],
                    sems.at[b],
                ).start()

    plsc.subcore_barrier()
    pltpu.sync_copy(
        acc_sh.at[pl.ds(s * _RPS, _RPS)],
        part_hbm.at[c, pl.ds(s * _RPS, _RPS)],
    )


def _add_body(p_ref, o_ref):
    o_ref[...] = p_ref[0] + p_ref[1]


def kernel(atom_features, node_graph_indices):
    ids2d = node_graph_indices.astype(jnp.int32).reshape(_NT, _TILE)
    ids2d = jnp.pad(ids2d, ((0, _IDS_PAD - _NT), (0, 0)))
    mesh = plsc.VectorSubcoreMesh(core_axis_name="c", subcore_axis_name="s")
    sc_call = pl.kernel(
        _sc_body,
        out_type=jax.ShapeDtypeStruct((_NC, _NSEG, _D), jnp.float32),
        mesh=mesh,
        scratch_types=[
            pltpu.VMEM((_TPW, _TILE), jnp.int32),
            pltpu.VMEM((_NBUF, _TILE, _D), jnp.float32),
            pltpu.VMEM_SHARED((_NSEG, _D), jnp.float32),
            pltpu.SemaphoreType.DMA((_NBUF,)),
        ],
    )
    part = sc_call(atom_features, ids2d)
    return pl.pallas_call(
        _add_body,
        out_shape=jax.ShapeDtypeStruct((_NSEG, _D), jnp.float32),
    )(part)
